# Initial kernel scaffold; baseline (speedup 1.0000x reference)
import jax
import jax.numpy as jnp
from jax.experimental import pallas as pl


def kernel(x, wg, w1, b1, w2, b2):
    raise NotImplementedError("write your pallas kernel here")


# speedup vs baseline: 1.0476x; 1.0476x over previous
import functools

import jax
import jax.numpy as jnp
from jax import lax
from jax.experimental import pallas as pl
from jax.experimental.pallas import tpu as pltpu
from jax.experimental.pallas import tpu_sc as plsc

S = 2048
M = 1024
E = 8
C = 512
F = 2048
NCHUNK = 8
CHW = C // NCHUNK
NSLOT = E * C
PAD_TOK = S
TOK_TAB = 4112
DUMP = 4096


def _gating_body(x_ref, wg_ref, meta_ref, laux_ref):
    x = x_ref[...]
    wg = wg_ref[...]
    logits = jnp.dot(x, wg, preferred_element_type=jnp.float32)

    mx = jnp.max(logits, axis=1, keepdims=True)
    ex = jnp.exp(logits - mx)
    gates = ex / jnp.sum(ex, axis=1, keepdims=True)

    col = lax.broadcasted_iota(jnp.int32, (S, E), 1)

    gmax = jnp.max(gates, axis=1, keepdims=True)
    idx1 = jnp.min(jnp.where(gates == gmax, col, E), axis=1, keepdims=True)
    mask1 = (col == idx1).astype(jnp.float32)

    l2 = jnp.where(col == idx1, -1e30, logits)
    l2max = jnp.max(l2, axis=1, keepdims=True)
    idx2 = jnp.min(jnp.where(l2 == l2max, col, E), axis=1, keepdims=True)
    mask2 = (col == idx2).astype(jnp.float32)

    ri = lax.broadcasted_iota(jnp.int32, (S, S), 0)
    ci = lax.broadcasted_iota(jnp.int32, (S, S), 1)
    tri = (ci < ri).astype(jnp.float32)
    loc1 = jnp.dot(tri, mask1, preferred_element_type=jnp.float32)
    n1 = jnp.sum(mask1, axis=0, keepdims=True)
    loc2 = jnp.dot(tri, mask2, preferred_element_type=jnp.float32) + n1

    me = jnp.sum(gates, axis=0, keepdims=True) / S
    ce = jnp.sum(mask1, axis=0, keepdims=True) / S
    laux_ref[...] = jnp.sum(me * ce, axis=1, keepdims=True) * E

    keep1 = (loc1 < C).astype(jnp.float32) * mask1
    keep2 = (loc2 < C).astype(jnp.float32) * mask2

    c1 = jnp.sum(loc1 * keep1, axis=1, keepdims=True).astype(jnp.int32)
    c2 = jnp.sum(loc2 * keep2, axis=1, keepdims=True).astype(jnp.int32)
    valid1 = jnp.sum(keep1, axis=1, keepdims=True) > 0.0
    valid2 = jnp.sum(keep2, axis=1, keepdims=True) > 0.0

    g1s = jnp.sum(gates * keep1, axis=1, keepdims=True)
    g2s = jnp.sum(gates * keep2, axis=1, keepdims=True)
    denom = jnp.maximum(g1s + g2s, jnp.finfo(jnp.float32).eps)
    g1 = g1s / denom
    g2 = g2s / denom

    def grow(e_idx, c_idx, valid):
        g = ((c_idx >> 6) << 9) + (e_idx << 6) + (c_idx & 63)
        return jnp.where(valid, g, DUMP)

    gs1 = grow(idx1, c1, valid1)
    gs2 = grow(idx2, c2, valid2)
    r1 = jnp.minimum(gs1, NSLOT - 1)
    r2 = jnp.minimum(gs2, NSLOT - 1)

    zero = jnp.zeros((S, 1), jnp.float32)
    meta = jnp.concatenate(
        [gs1.astype(jnp.float32), gs2.astype(jnp.float32),
         r1.astype(jnp.float32), r2.astype(jnp.float32),
         g1, g2, zero, zero], axis=1)
    meta_ref[...] = meta


def _gating(x2d, wg):
    return pl.pallas_call(
        _gating_body,
        out_shape=(jax.ShapeDtypeStruct((S, E), jnp.float32),
                   jax.ShapeDtypeStruct((1, 1), jnp.float32)),
    )(x2d, wg)


def _scatter_body(gs1_hbm, gs2_hbm, tok_hbm, tok_v, i1_v, i2_v):
    cid = lax.axis_index("c")
    sid = lax.axis_index("s")

    @pl.when(jnp.logical_and(cid == 0, sid == 0))
    def _():
        def init(i, _):
            tok_v[pl.ds(i * 16, 16)] = jnp.full((16,), PAD_TOK, jnp.int32)
            return 0
        lax.fori_loop(0, TOK_TAB // 16, init, 0)

        pltpu.sync_copy(gs1_hbm, i1_v)
        pltpu.sync_copy(gs2_hbm, i2_v)

        base = lax.iota(jnp.int32, 16)

        def scat(i, _):
            toks = base + i * 16
            plsc.store_scatter(tok_v, [i1_v[pl.ds(i * 16, 16)]], toks)
            plsc.store_scatter(tok_v, [i2_v[pl.ds(i * 16, 16)]], toks)
            return 0
        lax.fori_loop(0, S // 16, scat, 0)

        pltpu.sync_copy(tok_v, tok_hbm)


def _scatter_tokens(gs1, gs2):
    mesh = plsc.VectorSubcoreMesh(core_axis_name="c", subcore_axis_name="s")
    return pl.kernel(
        _scatter_body,
        mesh=mesh,
        out_type=jax.ShapeDtypeStruct((TOK_TAB,), jnp.int32),
        compiler_params=pltpu.CompilerParams(needs_layout_passes=False),
        scratch_types=[
            pltpu.VMEM((TOK_TAB,), jnp.int32),
            pltpu.VMEM((S,), jnp.int32),
            pltpu.VMEM((S,), jnp.int32),
        ],
    )(gs1, gs2)


def _gather_body(nrows, ncols, table_hbm, idx_hbm, out_hbm, idx_v, rows_v, sem):
    cid = lax.axis_index("c")
    sid = lax.axis_index("s")
    wid = sid * 2 + cid
    per_w = nrows // 32
    chunk = 32
    for ci in range(per_w // chunk):
        base = wid * per_w + ci * chunk
        pltpu.sync_copy(idx_hbm.at[pl.ds(base, chunk)], idx_v)
        pltpu.async_copy(table_hbm.at[idx_v], rows_v, sem).wait()
        pltpu.sync_copy(rows_v, out_hbm.at[pl.ds(base, chunk)])


def _gather_rows(table, idx, nrows, ncols):
    mesh = plsc.VectorSubcoreMesh(core_axis_name="c", subcore_axis_name="s")
    body = functools.partial(_gather_body, nrows, ncols)
    return pl.kernel(
        body,
        mesh=mesh,
        out_type=jax.ShapeDtypeStruct((nrows, ncols), jnp.float32),
        scratch_types=[
            pltpu.VMEM((32,), jnp.int32),
            pltpu.VMEM((32, ncols), jnp.float32),
            pltpu.SemaphoreType.DMA,
        ],
    )(table, idx)


def _ffn_body(x_ref, w1_ref, b1_ref, w2_ref, b2_ref, o_ref):
    h = jnp.dot(x_ref[...], w1_ref[0], preferred_element_type=jnp.float32)
    h = jnp.maximum(h + b1_ref[0], 0.0)
    o = jnp.dot(h, w2_ref[0], preferred_element_type=jnp.float32)
    o_ref[...] = o + b2_ref[0]


def _ffn(xg, w1, b1, w2, b2):
    return pl.pallas_call(
        _ffn_body,
        grid=(NCHUNK,),
        in_specs=[
            pl.BlockSpec((C, M), lambda j: (j, 0)),
            pl.BlockSpec((1, M, F), lambda j: (j, 0, 0)),
            pl.BlockSpec((1, 1, F), lambda j: (j, 0, 0)),
            pl.BlockSpec((1, F, M), lambda j: (j, 0, 0)),
            pl.BlockSpec((1, 1, M), lambda j: (j, 0, 0)),
        ],
        out_specs=pl.BlockSpec((C, M), lambda j: (j, 0)),
        out_shape=jax.ShapeDtypeStruct((NSLOT, M), jnp.float32),
        compiler_params=pltpu.CompilerParams(
            dimension_semantics=("arbitrary",)),
    )(xg, w1, b1, w2, b2)


def _combine_body(r1_ref, r2_ref, g1_ref, g2_ref, o_ref):
    o_ref[...] = g1_ref[...] * r1_ref[...] + g2_ref[...] * r2_ref[...]


def _combine(rows1, rows2, g1, g2):
    blk = 512
    return pl.pallas_call(
        _combine_body,
        grid=(S // blk,),
        in_specs=[
            pl.BlockSpec((blk, M), lambda i: (i, 0)),
            pl.BlockSpec((blk, M), lambda i: (i, 0)),
            pl.BlockSpec((blk, 1), lambda i: (i, 0)),
            pl.BlockSpec((blk, 1), lambda i: (i, 0)),
        ],
        out_specs=pl.BlockSpec((blk, M), lambda i: (i, 0)),
        out_shape=jax.ShapeDtypeStruct((S, M), jnp.float32),
        compiler_params=pltpu.CompilerParams(
            dimension_semantics=("parallel",)),
    )(rows1, rows2, g1, g2)


def kernel(x, wg, w1, b1, w2, b2):
    B = x.shape[0]
    x2d = x.reshape(S, M)

    meta, laux = _gating(x2d, wg)
    gs1 = meta[:, 0].astype(jnp.int32)
    gs2 = meta[:, 1].astype(jnp.int32)
    r1 = meta[:, 2].astype(jnp.int32)
    r2 = meta[:, 3].astype(jnp.int32)
    g1 = meta[:, 4:5]
    g2 = meta[:, 5:6]

    tok = _scatter_tokens(gs1, gs2)[:NSLOT]

    x_pad = jnp.concatenate(
        [x2d, jnp.zeros((8, M), jnp.float32)], axis=0)
    xg = _gather_rows(x_pad, tok, NSLOT, M)

    eo = _ffn(xg, w1, b1.reshape(E, 1, F), w2, b2.reshape(E, 1, M))

    rows = _gather_rows(eo, jnp.concatenate([r1, r2]), 2 * S, M)
    out = _combine(rows[:S], rows[S:], g1, g2)

    return out.reshape(B, S, M), laux.reshape(())


# speedup vs baseline: 1.0656x; 1.0172x over previous
import functools

import jax
import jax.numpy as jnp
from jax import lax
from jax.experimental import pallas as pl
from jax.experimental.pallas import tpu as pltpu
from jax.experimental.pallas import tpu_sc as plsc

S = 2048
M = 1024
E = 8
C = 512
F = 2048
NCHUNK = 8
CHW = C // NCHUNK
NSLOT = E * C
PAD_TOK = S
TOK_TAB = 4112
DUMP = 4096


def _gating_body(x_ref, wg_ref, meta_ref, laux_ref):
    x = x_ref[...]
    wg = wg_ref[...]
    logits = jnp.dot(x, wg, preferred_element_type=jnp.float32)

    mx = jnp.max(logits, axis=1, keepdims=True)
    ex = jnp.exp(logits - mx)
    gates = ex / jnp.sum(ex, axis=1, keepdims=True)

    col = lax.broadcasted_iota(jnp.int32, (S, E), 1)

    gmax = jnp.max(gates, axis=1, keepdims=True)
    idx1 = jnp.min(jnp.where(gates == gmax, col, E), axis=1, keepdims=True)
    mask1 = (col == idx1).astype(jnp.float32)

    l2 = jnp.where(col == idx1, -1e30, logits)
    l2max = jnp.max(l2, axis=1, keepdims=True)
    idx2 = jnp.min(jnp.where(l2 == l2max, col, E), axis=1, keepdims=True)
    mask2 = (col == idx2).astype(jnp.float32)

    ri = lax.broadcasted_iota(jnp.int32, (S, S), 0)
    ci = lax.broadcasted_iota(jnp.int32, (S, S), 1)
    tri = (ci < ri).astype(jnp.float32)
    loc1 = jnp.dot(tri, mask1, preferred_element_type=jnp.float32)
    n1 = jnp.sum(mask1, axis=0, keepdims=True)
    loc2 = jnp.dot(tri, mask2, preferred_element_type=jnp.float32) + n1

    me = jnp.sum(gates, axis=0, keepdims=True) / S
    ce = jnp.sum(mask1, axis=0, keepdims=True) / S
    laux_ref[...] = jnp.sum(me * ce, axis=1, keepdims=True) * E

    keep1 = (loc1 < C).astype(jnp.float32) * mask1
    keep2 = (loc2 < C).astype(jnp.float32) * mask2

    c1 = jnp.sum(loc1 * keep1, axis=1, keepdims=True).astype(jnp.int32)
    c2 = jnp.sum(loc2 * keep2, axis=1, keepdims=True).astype(jnp.int32)
    valid1 = jnp.sum(keep1, axis=1, keepdims=True) > 0.0
    valid2 = jnp.sum(keep2, axis=1, keepdims=True) > 0.0

    g1s = jnp.sum(gates * keep1, axis=1, keepdims=True)
    g2s = jnp.sum(gates * keep2, axis=1, keepdims=True)
    denom = jnp.maximum(g1s + g2s, jnp.finfo(jnp.float32).eps)
    g1 = g1s / denom
    g2 = g2s / denom

    def grow(e_idx, c_idx, valid):
        g = ((c_idx >> 6) << 9) + (e_idx << 6) + (c_idx & 63)
        return jnp.where(valid, g, DUMP)

    gs1 = grow(idx1, c1, valid1)
    gs2 = grow(idx2, c2, valid2)
    r1 = jnp.minimum(gs1, NSLOT - 1)
    r2 = jnp.minimum(gs2, NSLOT - 1)

    zero = jnp.zeros((S, 1), jnp.float32)
    meta = jnp.concatenate(
        [gs1.astype(jnp.float32), gs2.astype(jnp.float32),
         r1.astype(jnp.float32), r2.astype(jnp.float32),
         g1, g2, zero, zero], axis=1)
    meta_ref[...] = meta


def _gating(x2d, wg):
    return pl.pallas_call(
        _gating_body,
        out_shape=(jax.ShapeDtypeStruct((S, E), jnp.float32),
                   jax.ShapeDtypeStruct((1, 1), jnp.float32)),
    )(x2d, wg)


def _scatter_body(gs1_hbm, gs2_hbm, tok_hbm, tok_v, i1_v, i2_v):
    cid = lax.axis_index("c")
    sid = lax.axis_index("s")

    @pl.when(jnp.logical_and(cid == 0, sid == 0))
    def _():
        def init(i, _):
            tok_v[pl.ds(i * 16, 16)] = jnp.full((16,), PAD_TOK, jnp.int32)
            return 0
        lax.fori_loop(0, TOK_TAB // 16, init, 0)

        pltpu.sync_copy(gs1_hbm, i1_v)
        pltpu.sync_copy(gs2_hbm, i2_v)

        base = lax.iota(jnp.int32, 16)

        def scat(i, _):
            toks = base + i * 16
            plsc.store_scatter(tok_v, [i1_v[pl.ds(i * 16, 16)]], toks)
            plsc.store_scatter(tok_v, [i2_v[pl.ds(i * 16, 16)]], toks)
            return 0
        lax.fori_loop(0, S // 16, scat, 0)

        pltpu.sync_copy(tok_v, tok_hbm)


def _scatter_tokens(gs1, gs2):
    mesh = plsc.VectorSubcoreMesh(core_axis_name="c", subcore_axis_name="s")
    return pl.kernel(
        _scatter_body,
        mesh=mesh,
        out_type=jax.ShapeDtypeStruct((TOK_TAB,), jnp.int32),
        compiler_params=pltpu.CompilerParams(needs_layout_passes=False),
        scratch_types=[
            pltpu.VMEM((TOK_TAB,), jnp.int32),
            pltpu.VMEM((S,), jnp.int32),
            pltpu.VMEM((S,), jnp.int32),
        ],
    )(gs1, gs2)


def _gather_body(nrows, ncols, table_hbm, idx_hbm, out_hbm,
                 idx_v, rows0_v, rows1_v, gsem, osem0, osem1):
    cid = lax.axis_index("c")
    sid = lax.axis_index("s")
    wid = sid * 2 + cid
    per_w = nrows // 32
    chunk = 32
    nch = per_w // chunk
    rows = (rows0_v, rows1_v)
    osems = (osem0, osem1)
    pltpu.sync_copy(idx_hbm.at[pl.ds(wid * per_w, per_w)], idx_v)
    copies = [None, None]
    for ci in range(nch):
        b = ci % 2
        base = wid * per_w + ci * chunk
        if copies[b] is not None:
            copies[b].wait()
        pltpu.async_copy(
            table_hbm.at[idx_v.at[pl.ds(ci * chunk, chunk)]],
            rows[b], gsem).wait()
        copies[b] = pltpu.async_copy(
            rows[b], out_hbm.at[pl.ds(base, chunk)], osems[b])
    for c in copies:
        if c is not None:
            c.wait()


def _gather_rows(table, idx, nrows, ncols):
    mesh = plsc.VectorSubcoreMesh(core_axis_name="c", subcore_axis_name="s")
    body = functools.partial(_gather_body, nrows, ncols)
    return pl.kernel(
        body,
        mesh=mesh,
        out_type=jax.ShapeDtypeStruct((nrows, ncols), jnp.float32),
        scratch_types=[
            pltpu.VMEM((nrows // 32,), jnp.int32),
            pltpu.VMEM((32, ncols), jnp.float32),
            pltpu.VMEM((32, ncols), jnp.float32),
            pltpu.SemaphoreType.DMA,
            pltpu.SemaphoreType.DMA,
            pltpu.SemaphoreType.DMA,
        ],
    )(table, idx)


def _ffn_body(x_ref, w1_ref, b1_ref, w2_ref, b2_ref, o_ref):
    h = jnp.dot(x_ref[...], w1_ref[0], preferred_element_type=jnp.float32)
    h = jnp.maximum(h + b1_ref[0], 0.0)
    o = jnp.dot(h, w2_ref[0], preferred_element_type=jnp.float32)
    o_ref[...] = o + b2_ref[0]


def _ffn(xg, w1, b1, w2, b2):
    return pl.pallas_call(
        _ffn_body,
        grid=(NCHUNK,),
        in_specs=[
            pl.BlockSpec((C, M), lambda j: (j, 0)),
            pl.BlockSpec((1, M, F), lambda j: (j, 0, 0)),
            pl.BlockSpec((1, 1, F), lambda j: (j, 0, 0)),
            pl.BlockSpec((1, F, M), lambda j: (j, 0, 0)),
            pl.BlockSpec((1, 1, M), lambda j: (j, 0, 0)),
        ],
        out_specs=pl.BlockSpec((C, M), lambda j: (j, 0)),
        out_shape=jax.ShapeDtypeStruct((NSLOT, M), jnp.float32),
        compiler_params=pltpu.CompilerParams(
            dimension_semantics=("arbitrary",)),
    )(xg, w1, b1, w2, b2)


def _combine_body(r1_ref, r2_ref, g1_ref, g2_ref, o_ref):
    o_ref[...] = g1_ref[...] * r1_ref[...] + g2_ref[...] * r2_ref[...]


def _combine(rows1, rows2, g1, g2):
    blk = 512
    return pl.pallas_call(
        _combine_body,
        grid=(S // blk,),
        in_specs=[
            pl.BlockSpec((blk, M), lambda i: (i, 0)),
            pl.BlockSpec((blk, M), lambda i: (i, 0)),
            pl.BlockSpec((blk, 1), lambda i: (i, 0)),
            pl.BlockSpec((blk, 1), lambda i: (i, 0)),
        ],
        out_specs=pl.BlockSpec((blk, M), lambda i: (i, 0)),
        out_shape=jax.ShapeDtypeStruct((S, M), jnp.float32),
        compiler_params=pltpu.CompilerParams(
            dimension_semantics=("parallel",)),
    )(rows1, rows2, g1, g2)


def kernel(x, wg, w1, b1, w2, b2):
    B = x.shape[0]
    x2d = x.reshape(S, M)

    meta, laux = _gating(x2d, wg)
    gs1 = meta[:, 0].astype(jnp.int32)
    gs2 = meta[:, 1].astype(jnp.int32)
    r1 = meta[:, 2].astype(jnp.int32)
    r2 = meta[:, 3].astype(jnp.int32)
    g1 = meta[:, 4:5]
    g2 = meta[:, 5:6]

    tok = _scatter_tokens(gs1, gs2)[:NSLOT]

    x_pad = jnp.concatenate(
        [x2d, jnp.zeros((8, M), jnp.float32)], axis=0)
    xg = _gather_rows(x_pad, tok, NSLOT, M)

    eo = _ffn(xg, w1, b1.reshape(E, 1, F), w2, b2.reshape(E, 1, M))

    rows = _gather_rows(eo, jnp.concatenate([r1, r2]), 2 * S, M)
    out = _combine(rows[:S], rows[S:], g1, g2)

    return out.reshape(B, S, M), laux.reshape(())


# speedup vs baseline: 1.1886x; 1.1155x over previous
import functools

import jax
import jax.numpy as jnp
from jax import lax
from jax.experimental import pallas as pl
from jax.experimental.pallas import tpu as pltpu
from jax.experimental.pallas import tpu_sc as plsc

S = 2048
M = 1024
E = 8
C = 512
F = 2048
NCHUNK = 8
CHW = C // NCHUNK
NSLOT = E * C
PAD_TOK = S
TOK_TAB = 4112
DUMP = 4096


def _gating_body(x_ref, wg_ref, meta_ref, laux_ref):
    x = x_ref[...]
    wg = wg_ref[...]
    logits = jnp.dot(x, wg, preferred_element_type=jnp.float32)

    mx = jnp.max(logits, axis=1, keepdims=True)
    ex = jnp.exp(logits - mx)
    gates = ex / jnp.sum(ex, axis=1, keepdims=True)

    col = lax.broadcasted_iota(jnp.int32, (S, E), 1)

    gmax = jnp.max(gates, axis=1, keepdims=True)
    idx1 = jnp.min(jnp.where(gates == gmax, col, E), axis=1, keepdims=True)
    mask1 = (col == idx1).astype(jnp.float32)

    l2 = jnp.where(col == idx1, -1e30, logits)
    l2max = jnp.max(l2, axis=1, keepdims=True)
    idx2 = jnp.min(jnp.where(l2 == l2max, col, E), axis=1, keepdims=True)
    mask2 = (col == idx2).astype(jnp.float32)

    ri = lax.broadcasted_iota(jnp.int32, (S, S), 0)
    ci = lax.broadcasted_iota(jnp.int32, (S, S), 1)
    tri = (ci < ri).astype(jnp.float32)
    loc1 = jnp.dot(tri, mask1, preferred_element_type=jnp.float32)
    n1 = jnp.sum(mask1, axis=0, keepdims=True)
    loc2 = jnp.dot(tri, mask2, preferred_element_type=jnp.float32) + n1

    me = jnp.sum(gates, axis=0, keepdims=True) / S
    ce = jnp.sum(mask1, axis=0, keepdims=True) / S
    laux_ref[...] = jnp.sum(me * ce, axis=1, keepdims=True) * E

    keep1 = (loc1 < C).astype(jnp.float32) * mask1
    keep2 = (loc2 < C).astype(jnp.float32) * mask2

    c1 = jnp.sum(loc1 * keep1, axis=1, keepdims=True).astype(jnp.int32)
    c2 = jnp.sum(loc2 * keep2, axis=1, keepdims=True).astype(jnp.int32)
    valid1 = jnp.sum(keep1, axis=1, keepdims=True) > 0.0
    valid2 = jnp.sum(keep2, axis=1, keepdims=True) > 0.0

    g1s = jnp.sum(gates * keep1, axis=1, keepdims=True)
    g2s = jnp.sum(gates * keep2, axis=1, keepdims=True)
    denom = jnp.maximum(g1s + g2s, jnp.finfo(jnp.float32).eps)
    g1 = g1s / denom
    g2 = g2s / denom

    def grow(e_idx, c_idx, valid):
        g = ((c_idx >> 6) << 9) + (e_idx << 6) + (c_idx & 63)
        return jnp.where(valid, g, DUMP)

    gs1 = grow(idx1, c1, valid1)
    gs2 = grow(idx2, c2, valid2)
    r1 = jnp.minimum(gs1, NSLOT - 1)
    r2 = jnp.minimum(gs2, NSLOT - 1)

    zero = jnp.zeros((S, 1), jnp.float32)
    meta = jnp.concatenate(
        [gs1.astype(jnp.float32), gs2.astype(jnp.float32),
         r1.astype(jnp.float32), r2.astype(jnp.float32),
         g1, g2, zero, zero], axis=1)
    meta_ref[...] = meta


def _gating(x2d, wg):
    return pl.pallas_call(
        _gating_body,
        out_shape=(jax.ShapeDtypeStruct((S, E), jnp.float32),
                   jax.ShapeDtypeStruct((1, 1), jnp.float32)),
    )(x2d, wg)


def _dispatch_body(x_hbm, sidx_hbm, out_hbm, rows_v, i1_v, i2_v, sem):
    cid = lax.axis_index("c")
    sid = lax.axis_index("s")
    wid = sid * 2 + cid
    base = wid * (S // 32)
    pltpu.sync_copy(sidx_hbm.at[wid, 0], i1_v)
    pltpu.sync_copy(sidx_hbm.at[wid, 1], i2_v)
    pltpu.sync_copy(x_hbm.at[pl.ds(base, S // 32)], rows_v)
    c1 = pltpu.async_copy(rows_v, out_hbm.at[i1_v], sem)
    c2 = pltpu.async_copy(rows_v, out_hbm.at[i2_v], sem)
    c1.wait()
    c2.wait()


def _dispatch_scatter(x2d, sidx):
    mesh = plsc.VectorSubcoreMesh(core_axis_name="c", subcore_axis_name="s")
    return pl.kernel(
        _dispatch_body,
        mesh=mesh,
        out_type=jax.ShapeDtypeStruct((TOK_TAB // 1, M), jnp.float32),
        scratch_types=[
            pltpu.VMEM((S // 32, M), jnp.float32),
            pltpu.VMEM((S // 32,), jnp.int32),
            pltpu.VMEM((S // 32,), jnp.int32),
            pltpu.SemaphoreType.DMA,
        ],
    )(x2d, sidx)


def _gather_body(nrows, ncols, table_hbm, idx_hbm, out_hbm,
                 idx_v, rows0_v, rows1_v, gsem, osem0, osem1):
    cid = lax.axis_index("c")
    sid = lax.axis_index("s")
    wid = sid * 2 + cid
    per_w = nrows // 32
    chunk = 32
    nch = per_w // chunk
    rows = (rows0_v, rows1_v)
    osems = (osem0, osem1)
    pltpu.sync_copy(idx_hbm.at[pl.ds(wid * per_w, per_w)], idx_v)
    copies = [None, None]
    for ci in range(nch):
        b = ci % 2
        base = wid * per_w + ci * chunk
        if copies[b] is not None:
            copies[b].wait()
        pltpu.async_copy(
            table_hbm.at[idx_v.at[pl.ds(ci * chunk, chunk)]],
            rows[b], gsem).wait()
        copies[b] = pltpu.async_copy(
            rows[b], out_hbm.at[pl.ds(base, chunk)], osems[b])
    for c in copies:
        if c is not None:
            c.wait()


def _gather_rows(table, idx, nrows, ncols):
    mesh = plsc.VectorSubcoreMesh(core_axis_name="c", subcore_axis_name="s")
    body = functools.partial(_gather_body, nrows, ncols)
    return pl.kernel(
        body,
        mesh=mesh,
        out_type=jax.ShapeDtypeStruct((nrows, ncols), jnp.float32),
        scratch_types=[
            pltpu.VMEM((nrows // 32,), jnp.int32),
            pltpu.VMEM((32, ncols), jnp.float32),
            pltpu.VMEM((32, ncols), jnp.float32),
            pltpu.SemaphoreType.DMA,
            pltpu.SemaphoreType.DMA,
            pltpu.SemaphoreType.DMA,
        ],
    )(table, idx)


def _ffn_body(x_ref, w1_ref, b1_ref, w2_ref, b2_ref, o_ref):
    h = jnp.dot(x_ref[...], w1_ref[0], preferred_element_type=jnp.float32)
    h = jnp.maximum(h + b1_ref[0], 0.0)
    o = jnp.dot(h, w2_ref[0], preferred_element_type=jnp.float32)
    o_ref[...] = o + b2_ref[0]


def _ffn(xg, w1, b1, w2, b2):
    return pl.pallas_call(
        _ffn_body,
        grid=(NCHUNK,),
        in_specs=[
            pl.BlockSpec((C, M), lambda j: (j, 0)),
            pl.BlockSpec((1, M, F), lambda j: (j, 0, 0)),
            pl.BlockSpec((1, 1, F), lambda j: (j, 0, 0)),
            pl.BlockSpec((1, F, M), lambda j: (j, 0, 0)),
            pl.BlockSpec((1, 1, M), lambda j: (j, 0, 0)),
        ],
        out_specs=pl.BlockSpec((C, M), lambda j: (j, 0)),
        out_shape=jax.ShapeDtypeStruct((NSLOT, M), jnp.float32),
        compiler_params=pltpu.CompilerParams(
            dimension_semantics=("arbitrary",)),
    )(xg, w1, b1, w2, b2)


def _combine_body(r1_ref, r2_ref, g1_ref, g2_ref, o_ref):
    g1 = g1_ref[...]
    g2 = g2_ref[...]
    t1 = jnp.where(g1 > 0.0, g1 * r1_ref[...], 0.0)
    t2 = jnp.where(g2 > 0.0, g2 * r2_ref[...], 0.0)
    o_ref[...] = t1 + t2


def _combine(rows1, rows2, g1, g2):
    blk = 512
    return pl.pallas_call(
        _combine_body,
        grid=(S // blk,),
        in_specs=[
            pl.BlockSpec((blk, M), lambda i: (i, 0)),
            pl.BlockSpec((blk, M), lambda i: (i, 0)),
            pl.BlockSpec((blk, 1), lambda i: (i, 0)),
            pl.BlockSpec((blk, 1), lambda i: (i, 0)),
        ],
        out_specs=pl.BlockSpec((blk, M), lambda i: (i, 0)),
        out_shape=jax.ShapeDtypeStruct((S, M), jnp.float32),
        compiler_params=pltpu.CompilerParams(
            dimension_semantics=("parallel",)),
    )(rows1, rows2, g1, g2)


def kernel(x, wg, w1, b1, w2, b2):
    B = x.shape[0]
    x2d = x.reshape(S, M)

    meta, laux = _gating(x2d, wg)
    gs1 = meta[:, 0].astype(jnp.int32)
    gs2 = meta[:, 1].astype(jnp.int32)
    r1 = meta[:, 2].astype(jnp.int32)
    r2 = meta[:, 3].astype(jnp.int32)
    g1 = meta[:, 4:5]
    g2 = meta[:, 5:6]

    sidx = jnp.stack([gs1, gs2]).reshape(2, 32, S // 32).transpose(1, 0, 2)
    xg = _dispatch_scatter(x2d, sidx)

    eo = _ffn(xg, w1, b1.reshape(E, 1, F), w2, b2.reshape(E, 1, M))

    rows = _gather_rows(eo, jnp.concatenate([r1, r2]), 2 * S, M)
    out = _combine(rows[:S], rows[S:], g1, g2)

    return out.reshape(B, S, M), laux.reshape(())


# speedup vs baseline: 1.3331x; 1.1215x over previous
import functools

import jax
import jax.numpy as jnp
from jax import lax
from jax.experimental import pallas as pl
from jax.experimental.pallas import tpu as pltpu
from jax.experimental.pallas import tpu_sc as plsc

S = 2048
M = 1024
E = 8
C = 512
F = 2048
NCHUNK = 8
CHW = C // NCHUNK
NSLOT = E * C
PAD_TOK = S
TOK_TAB = 4112
DUMP = 4096


def _gating_body(x_ref, wg_ref, meta_ref, laux_ref):
    x = x_ref[...]
    wg = wg_ref[...]
    logits = jnp.dot(x, wg, preferred_element_type=jnp.float32)

    mx = jnp.max(logits, axis=1, keepdims=True)
    ex = jnp.exp(logits - mx)
    gates = ex / jnp.sum(ex, axis=1, keepdims=True)

    col = lax.broadcasted_iota(jnp.int32, (S, E), 1)

    gmax = jnp.max(gates, axis=1, keepdims=True)
    idx1 = jnp.min(jnp.where(gates == gmax, col, E), axis=1, keepdims=True)
    mask1 = (col == idx1).astype(jnp.float32)

    l2 = jnp.where(col == idx1, -1e30, logits)
    l2max = jnp.max(l2, axis=1, keepdims=True)
    idx2 = jnp.min(jnp.where(l2 == l2max, col, E), axis=1, keepdims=True)
    mask2 = (col == idx2).astype(jnp.float32)

    ri = lax.broadcasted_iota(jnp.int32, (S, S), 0)
    ci = lax.broadcasted_iota(jnp.int32, (S, S), 1)
    tri = (ci < ri).astype(jnp.float32)
    loc1 = jnp.dot(tri, mask1, preferred_element_type=jnp.float32)
    n1 = jnp.sum(mask1, axis=0, keepdims=True)
    loc2 = jnp.dot(tri, mask2, preferred_element_type=jnp.float32) + n1

    me = jnp.sum(gates, axis=0, keepdims=True) / S
    ce = jnp.sum(mask1, axis=0, keepdims=True) / S
    laux_ref[...] = jnp.sum(me * ce, axis=1, keepdims=True) * E

    keep1 = (loc1 < C).astype(jnp.float32) * mask1
    keep2 = (loc2 < C).astype(jnp.float32) * mask2

    c1 = jnp.sum(loc1 * keep1, axis=1, keepdims=True).astype(jnp.int32)
    c2 = jnp.sum(loc2 * keep2, axis=1, keepdims=True).astype(jnp.int32)
    valid1 = jnp.sum(keep1, axis=1, keepdims=True) > 0.0
    valid2 = jnp.sum(keep2, axis=1, keepdims=True) > 0.0

    g1s = jnp.sum(gates * keep1, axis=1, keepdims=True)
    g2s = jnp.sum(gates * keep2, axis=1, keepdims=True)
    denom = jnp.maximum(g1s + g2s, jnp.finfo(jnp.float32).eps)
    g1 = g1s / denom
    g2 = g2s / denom

    def grow(e_idx, c_idx, valid):
        g = ((c_idx >> 6) << 9) + (e_idx << 6) + (c_idx & 63)
        return jnp.where(valid, g, DUMP)

    gs1 = grow(idx1, c1, valid1)
    gs2 = grow(idx2, c2, valid2)
    r1 = jnp.minimum(gs1, NSLOT - 1)
    r2 = jnp.minimum(gs2, NSLOT - 1)

    zero = jnp.zeros((S, 1), jnp.float32)
    meta = jnp.concatenate(
        [gs1.astype(jnp.float32), gs2.astype(jnp.float32),
         r1.astype(jnp.float32), r2.astype(jnp.float32),
         g1, g2, zero, zero], axis=1)
    meta_ref[...] = meta


def _gating(x2d, wg):
    return pl.pallas_call(
        _gating_body,
        out_shape=(jax.ShapeDtypeStruct((S, E), jnp.float32),
                   jax.ShapeDtypeStruct((1, 1), jnp.float32)),
    )(x2d, wg)


def _dispatch_body(x_hbm, sidx_hbm, out_hbm, rows_v, i1_v, i2_v, sem):
    cid = lax.axis_index("c")
    sid = lax.axis_index("s")
    wid = sid * 2 + cid
    base = wid * (S // 32)
    pltpu.sync_copy(sidx_hbm.at[wid, 0], i1_v)
    pltpu.sync_copy(sidx_hbm.at[wid, 1], i2_v)
    pltpu.sync_copy(x_hbm.at[pl.ds(base, S // 32)], rows_v)
    c1 = pltpu.async_copy(rows_v, out_hbm.at[i1_v], sem)
    c2 = pltpu.async_copy(rows_v, out_hbm.at[i2_v], sem)
    c1.wait()
    c2.wait()


def _dispatch_scatter(x2d, sidx):
    mesh = plsc.VectorSubcoreMesh(core_axis_name="c", subcore_axis_name="s")
    return pl.kernel(
        _dispatch_body,
        mesh=mesh,
        out_type=jax.ShapeDtypeStruct((TOK_TAB // 1, M), jnp.float32),
        scratch_types=[
            pltpu.VMEM((S // 32, M), jnp.float32),
            pltpu.VMEM((S // 32,), jnp.int32),
            pltpu.VMEM((S // 32,), jnp.int32),
            pltpu.SemaphoreType.DMA,
        ],
    )(x2d, sidx)


def _combine_body(eo_hbm, ridx_hbm, g1r_hbm, g2r_hbm, out_hbm,
                  i1_v, i2_v, g1_v, g2_v,
                  ra0, ra1, rb0, rb1, o0, o1,
                  gsem, osem0, osem1):
    cid = lax.axis_index("c")
    sid = lax.axis_index("s")
    wid = sid * 2 + cid
    ntok = S // 32
    ck = 16
    nch = ntok // ck
    base = wid * ntok
    pltpu.sync_copy(ridx_hbm.at[wid, 0], i1_v)
    pltpu.sync_copy(ridx_hbm.at[wid, 1], i2_v)
    pltpu.sync_copy(g1r_hbm.at[wid], g1_v)
    pltpu.sync_copy(g2r_hbm.at[wid], g2_v)
    ra = (ra0, ra1)
    rb = (rb0, rb1)
    ov = (o0, o1)
    osem = (osem0, osem1)

    def gathers(c, b):
        c1 = pltpu.async_copy(
            eo_hbm.at[i1_v.at[pl.ds(c * ck, ck)]], ra[b], gsem)
        c2 = pltpu.async_copy(
            eo_hbm.at[i2_v.at[pl.ds(c * ck, ck)]], rb[b], gsem)
        return c1, c2

    pend = gathers(0, 0)
    outc = [None, None]
    for c in range(nch):
        b = c % 2
        pend[0].wait()
        pend[1].wait()
        if c + 1 < nch:
            pend = gathers(c + 1, 1 - b)
        if outc[b] is not None:
            outc[b].wait()
        for t in range(ck):
            g1b = g1_v[c * ck + t]
            g2b = g2_v[c * ck + t]
            m1 = g1b > 0.0
            m2 = g2b > 0.0
            zero = jnp.zeros((16,), jnp.float32)

            def fma(k, _):
                off = k * 16
                t1 = jnp.where(m1, ra[b][t, pl.ds(off, 16)] * g1b, zero)
                t2 = jnp.where(m2, rb[b][t, pl.ds(off, 16)] * g2b, zero)
                ov[b][t, pl.ds(off, 16)] = t1 + t2
                return 0
            lax.fori_loop(0, M // 16, fma, 0)
        outc[b] = pltpu.async_copy(
            ov[b], out_hbm.at[pl.ds(base + c * ck, ck)], osem[b])
    for oc in outc:
        if oc is not None:
            oc.wait()


def _combine_sc(eo, ridx, g1r, g2r):
    mesh = plsc.VectorSubcoreMesh(core_axis_name="c", subcore_axis_name="s")
    ck = 16
    return pl.kernel(
        _combine_body,
        mesh=mesh,
        out_type=jax.ShapeDtypeStruct((S, M), jnp.float32),
        scratch_types=[
            pltpu.VMEM((S // 32,), jnp.int32),
            pltpu.VMEM((S // 32,), jnp.int32),
            pltpu.VMEM((S // 32, 16), jnp.float32),
            pltpu.VMEM((S // 32, 16), jnp.float32),
            pltpu.VMEM((ck, M), jnp.float32),
            pltpu.VMEM((ck, M), jnp.float32),
            pltpu.VMEM((ck, M), jnp.float32),
            pltpu.VMEM((ck, M), jnp.float32),
            pltpu.VMEM((ck, M), jnp.float32),
            pltpu.VMEM((ck, M), jnp.float32),
            pltpu.SemaphoreType.DMA,
            pltpu.SemaphoreType.DMA,
            pltpu.SemaphoreType.DMA,
        ],
    )(eo, ridx, g1r, g2r)


def _ffn_body(x_ref, w1_ref, b1_ref, w2_ref, b2_ref, o_ref):
    h = jnp.dot(x_ref[...], w1_ref[0], preferred_element_type=jnp.float32)
    h = jnp.maximum(h + b1_ref[0], 0.0)
    o = jnp.dot(h, w2_ref[0], preferred_element_type=jnp.float32)
    o_ref[...] = o + b2_ref[0]


def _ffn(xg, w1, b1, w2, b2):
    return pl.pallas_call(
        _ffn_body,
        grid=(NCHUNK,),
        in_specs=[
            pl.BlockSpec((C, M), lambda j: (j, 0)),
            pl.BlockSpec((1, M, F), lambda j: (j, 0, 0)),
            pl.BlockSpec((1, 1, F), lambda j: (j, 0, 0)),
            pl.BlockSpec((1, F, M), lambda j: (j, 0, 0)),
            pl.BlockSpec((1, 1, M), lambda j: (j, 0, 0)),
        ],
        out_specs=pl.BlockSpec((C, M), lambda j: (j, 0)),
        out_shape=jax.ShapeDtypeStruct((NSLOT, M), jnp.float32),
        compiler_params=pltpu.CompilerParams(
            dimension_semantics=("arbitrary",)),
    )(xg, w1, b1, w2, b2)


def kernel(x, wg, w1, b1, w2, b2):
    B = x.shape[0]
    x2d = x.reshape(S, M)

    meta, laux = _gating(x2d, wg)
    gs1 = meta[:, 0].astype(jnp.int32)
    gs2 = meta[:, 1].astype(jnp.int32)
    r1 = meta[:, 2].astype(jnp.int32)
    r2 = meta[:, 3].astype(jnp.int32)
    g1 = meta[:, 4:5]
    g2 = meta[:, 5:6]

    sidx = jnp.stack([gs1, gs2]).reshape(2, 32, S // 32).transpose(1, 0, 2)
    xg = _dispatch_scatter(x2d, sidx)

    eo = _ffn(xg, w1, b1.reshape(E, 1, F), w2, b2.reshape(E, 1, M))

    ridx = jnp.stack([r1, r2]).reshape(2, 32, S // 32).transpose(1, 0, 2)
    g1r = jnp.broadcast_to(g1, (S, 16)).reshape(32, S // 32, 16)
    g2r = jnp.broadcast_to(g2, (S, 16)).reshape(32, S // 32, 16)
    out = _combine_sc(eo, ridx, g1r, g2r)

    return out.reshape(B, S, M), laux.reshape(())


# speedup vs baseline: 1.3342x; 1.0008x over previous
import functools

import jax
import jax.numpy as jnp
from jax import lax
from jax.experimental import pallas as pl
from jax.experimental.pallas import tpu as pltpu
from jax.experimental.pallas import tpu_sc as plsc

S = 2048
M = 1024
E = 8
C = 512
F = 2048
NCHUNK = 8
CHW = C // NCHUNK
NSLOT = E * C
PAD_TOK = S
TOK_TAB = 4112
DUMP = 4096


def _gating_body(x_ref, wg_ref, meta_ref, laux_ref):
    x = x_ref[...]
    wg = wg_ref[...]
    logits = jnp.dot(x, wg, preferred_element_type=jnp.float32)

    mx = jnp.max(logits, axis=1, keepdims=True)
    ex = jnp.exp(logits - mx)
    gates = ex / jnp.sum(ex, axis=1, keepdims=True)

    col = lax.broadcasted_iota(jnp.int32, (S, E), 1)

    gmax = jnp.max(gates, axis=1, keepdims=True)
    idx1 = jnp.min(jnp.where(gates == gmax, col, E), axis=1, keepdims=True)
    mask1 = (col == idx1).astype(jnp.float32)

    l2 = jnp.where(col == idx1, -1e30, logits)
    l2max = jnp.max(l2, axis=1, keepdims=True)
    idx2 = jnp.min(jnp.where(l2 == l2max, col, E), axis=1, keepdims=True)
    mask2 = (col == idx2).astype(jnp.float32)

    ri = lax.broadcasted_iota(jnp.int32, (S, S), 0)
    ci = lax.broadcasted_iota(jnp.int32, (S, S), 1)
    tri = (ci < ri).astype(jnp.float32)
    loc1 = jnp.dot(tri, mask1, preferred_element_type=jnp.float32)
    n1 = jnp.sum(mask1, axis=0, keepdims=True)
    loc2 = jnp.dot(tri, mask2, preferred_element_type=jnp.float32) + n1

    me = jnp.sum(gates, axis=0, keepdims=True) / S
    ce = jnp.sum(mask1, axis=0, keepdims=True) / S
    laux_ref[...] = jnp.sum(me * ce, axis=1, keepdims=True) * E

    keep1 = (loc1 < C).astype(jnp.float32) * mask1
    keep2 = (loc2 < C).astype(jnp.float32) * mask2

    c1 = jnp.sum(loc1 * keep1, axis=1, keepdims=True).astype(jnp.int32)
    c2 = jnp.sum(loc2 * keep2, axis=1, keepdims=True).astype(jnp.int32)
    valid1 = jnp.sum(keep1, axis=1, keepdims=True) > 0.0
    valid2 = jnp.sum(keep2, axis=1, keepdims=True) > 0.0

    g1s = jnp.sum(gates * keep1, axis=1, keepdims=True)
    g2s = jnp.sum(gates * keep2, axis=1, keepdims=True)
    denom = jnp.maximum(g1s + g2s, jnp.finfo(jnp.float32).eps)
    g1 = g1s / denom
    g2 = g2s / denom

    def grow(e_idx, c_idx, valid):
        g = ((c_idx >> 6) << 9) + (e_idx << 6) + (c_idx & 63)
        return jnp.where(valid, g, DUMP)

    gs1 = grow(idx1, c1, valid1)
    gs2 = grow(idx2, c2, valid2)
    r1 = jnp.minimum(gs1, NSLOT - 1)
    r2 = jnp.minimum(gs2, NSLOT - 1)

    zero = jnp.zeros((S, 1), jnp.float32)
    meta = jnp.concatenate(
        [gs1.astype(jnp.float32), gs2.astype(jnp.float32),
         r1.astype(jnp.float32), r2.astype(jnp.float32),
         g1, g2, zero, zero], axis=1)
    meta_ref[...] = meta


def _gating(x2d, wg):
    return pl.pallas_call(
        _gating_body,
        out_shape=(jax.ShapeDtypeStruct((S, E), jnp.float32),
                   jax.ShapeDtypeStruct((1, 1), jnp.float32)),
    )(x2d, wg)


def _dispatch_body(x_hbm, sidx_hbm, out_hbm, rows_v, i1_v, i2_v, sem):
    cid = lax.axis_index("c")
    sid = lax.axis_index("s")
    wid = sid * 2 + cid
    base = wid * (S // 32)
    pltpu.sync_copy(sidx_hbm.at[wid, 0], i1_v)
    pltpu.sync_copy(sidx_hbm.at[wid, 1], i2_v)
    pltpu.sync_copy(x_hbm.at[pl.ds(base, S // 32)], rows_v)
    c1 = pltpu.async_copy(rows_v, out_hbm.at[i1_v], sem)
    c2 = pltpu.async_copy(rows_v, out_hbm.at[i2_v], sem)
    c1.wait()
    c2.wait()


def _dispatch_scatter(x2d, sidx):
    mesh = plsc.VectorSubcoreMesh(core_axis_name="c", subcore_axis_name="s")
    return pl.kernel(
        _dispatch_body,
        mesh=mesh,
        out_type=jax.ShapeDtypeStruct((TOK_TAB // 1, M), jnp.float32),
        scratch_types=[
            pltpu.VMEM((S // 32, M), jnp.float32),
            pltpu.VMEM((S // 32,), jnp.int32),
            pltpu.VMEM((S // 32,), jnp.int32),
            pltpu.SemaphoreType.DMA,
        ],
    )(x2d, sidx)


def _combine_body(eo_hbm, ridx_hbm, g1r_hbm, g2r_hbm, out_hbm,
                  i1_v, i2_v, g1_v, g2_v,
                  ra0, ra1, rb0, rb1, o0, o1,
                  gsem, osem0, osem1):
    cid = lax.axis_index("c")
    sid = lax.axis_index("s")
    wid = sid * 2 + cid
    ntok = S // 32
    ck = 16
    nch = ntok // ck
    base = wid * ntok
    pltpu.sync_copy(ridx_hbm.at[wid, 0], i1_v)
    pltpu.sync_copy(ridx_hbm.at[wid, 1], i2_v)
    pltpu.sync_copy(g1r_hbm.at[wid], g1_v)
    pltpu.sync_copy(g2r_hbm.at[wid], g2_v)
    ra = (ra0, ra1)
    rb = (rb0, rb1)
    ov = (o0, o1)
    osem = (osem0, osem1)

    def gathers(c, b):
        c1 = pltpu.async_copy(
            eo_hbm.at[i1_v.at[pl.ds(c * ck, ck)]], ra[b], gsem)
        c2 = pltpu.async_copy(
            eo_hbm.at[i2_v.at[pl.ds(c * ck, ck)]], rb[b], gsem)
        return c1, c2

    pend = gathers(0, 0)
    outc = [None, None]
    for c in range(nch):
        b = c % 2
        pend[0].wait()
        pend[1].wait()
        if c + 1 < nch:
            pend = gathers(c + 1, 1 - b)
        if outc[b] is not None:
            outc[b].wait()
        for t in range(ck):
            g1b = g1_v[c * ck + t]
            g2b = g2_v[c * ck + t]
            m1 = g1b > 0.0
            m2 = g2b > 0.0
            zero = jnp.zeros((16,), jnp.float32)

            def fma(k, _):
                for kk in range(8):
                    off = k * 128 + kk * 16
                    t1 = jnp.where(m1, ra[b][t, pl.ds(off, 16)] * g1b, zero)
                    t2 = jnp.where(m2, rb[b][t, pl.ds(off, 16)] * g2b, zero)
                    ov[b][t, pl.ds(off, 16)] = t1 + t2
                return 0
            lax.fori_loop(0, M // 128, fma, 0)
        outc[b] = pltpu.async_copy(
            ov[b], out_hbm.at[pl.ds(base + c * ck, ck)], osem[b])
    for oc in outc:
        if oc is not None:
            oc.wait()


def _combine_sc(eo, ridx, g1r, g2r):
    mesh = plsc.VectorSubcoreMesh(core_axis_name="c", subcore_axis_name="s")
    ck = 16
    return pl.kernel(
        _combine_body,
        mesh=mesh,
        out_type=jax.ShapeDtypeStruct((S, M), jnp.float32),
        scratch_types=[
            pltpu.VMEM((S // 32,), jnp.int32),
            pltpu.VMEM((S // 32,), jnp.int32),
            pltpu.VMEM((S // 32, 16), jnp.float32),
            pltpu.VMEM((S // 32, 16), jnp.float32),
            pltpu.VMEM((ck, M), jnp.float32),
            pltpu.VMEM((ck, M), jnp.float32),
            pltpu.VMEM((ck, M), jnp.float32),
            pltpu.VMEM((ck, M), jnp.float32),
            pltpu.VMEM((ck, M), jnp.float32),
            pltpu.VMEM((ck, M), jnp.float32),
            pltpu.SemaphoreType.DMA,
            pltpu.SemaphoreType.DMA,
            pltpu.SemaphoreType.DMA,
        ],
    )(eo, ridx, g1r, g2r)


def _ffn_body(x_ref, w1_ref, b1_ref, w2_ref, b2_ref, o_ref):
    h = jnp.dot(x_ref[...], w1_ref[0], preferred_element_type=jnp.float32)
    h = jnp.maximum(h + b1_ref[0], 0.0)
    o = jnp.dot(h, w2_ref[0], preferred_element_type=jnp.float32)
    o_ref[...] = o + b2_ref[0]


def _ffn(xg, w1, b1, w2, b2):
    return pl.pallas_call(
        _ffn_body,
        grid=(NCHUNK,),
        in_specs=[
            pl.BlockSpec((C, M), lambda j: (j, 0)),
            pl.BlockSpec((1, M, F), lambda j: (j, 0, 0)),
            pl.BlockSpec((1, 1, F), lambda j: (j, 0, 0)),
            pl.BlockSpec((1, F, M), lambda j: (j, 0, 0)),
            pl.BlockSpec((1, 1, M), lambda j: (j, 0, 0)),
        ],
        out_specs=pl.BlockSpec((C, M), lambda j: (j, 0)),
        out_shape=jax.ShapeDtypeStruct((NSLOT, M), jnp.float32),
        compiler_params=pltpu.CompilerParams(
            dimension_semantics=("arbitrary",)),
    )(xg, w1, b1, w2, b2)


def kernel(x, wg, w1, b1, w2, b2):
    B = x.shape[0]
    x2d = x.reshape(S, M)

    meta, laux = _gating(x2d, wg)
    gs1 = meta[:, 0].astype(jnp.int32)
    gs2 = meta[:, 1].astype(jnp.int32)
    r1 = meta[:, 2].astype(jnp.int32)
    r2 = meta[:, 3].astype(jnp.int32)
    g1 = meta[:, 4:5]
    g2 = meta[:, 5:6]

    sidx = jnp.stack([gs1, gs2]).reshape(2, 32, S // 32).transpose(1, 0, 2)
    xg = _dispatch_scatter(x2d, sidx)

    eo = _ffn(xg, w1, b1.reshape(E, 1, F), w2, b2.reshape(E, 1, M))

    ridx = jnp.stack([r1, r2]).reshape(2, 32, S // 32).transpose(1, 0, 2)
    g1r = jnp.broadcast_to(g1, (S, 16)).reshape(32, S // 32, 16)
    g2r = jnp.broadcast_to(g2, (S, 16)).reshape(32, S // 32, 16)
    out = _combine_sc(eo, ridx, g1r, g2r)

    return out.reshape(B, S, M), laux.reshape(())


# speedup vs baseline: 1.3634x; 1.0219x over previous
import functools

import jax
import jax.numpy as jnp
from jax import lax
from jax.experimental import pallas as pl
from jax.experimental.pallas import tpu as pltpu
from jax.experimental.pallas import tpu_sc as plsc

S = 2048
M = 1024
E = 8
C = 512
F = 2048
NCHUNK = 8
CHW = C // NCHUNK
NSLOT = E * C
PAD_TOK = S
TOK_TAB = 4112
DUMP = 4096


def _gating_body(x_ref, wg_ref, meta_ref, laux_ref):
    x = x_ref[...]
    wg = wg_ref[...]
    logits = jnp.dot(x, wg, preferred_element_type=jnp.float32)

    mx = jnp.max(logits, axis=1, keepdims=True)
    ex = jnp.exp(logits - mx)
    gates = ex / jnp.sum(ex, axis=1, keepdims=True)

    col = lax.broadcasted_iota(jnp.int32, (S, E), 1)

    gmax = jnp.max(gates, axis=1, keepdims=True)
    idx1 = jnp.min(jnp.where(gates == gmax, col, E), axis=1, keepdims=True)
    mask1 = (col == idx1).astype(jnp.float32)

    l2 = jnp.where(col == idx1, -1e30, logits)
    l2max = jnp.max(l2, axis=1, keepdims=True)
    idx2 = jnp.min(jnp.where(l2 == l2max, col, E), axis=1, keepdims=True)
    mask2 = (col == idx2).astype(jnp.float32)

    ri = lax.broadcasted_iota(jnp.int32, (S, S), 0)
    ci = lax.broadcasted_iota(jnp.int32, (S, S), 1)
    tri = (ci < ri).astype(jnp.float32)
    loc1 = jnp.dot(tri, mask1, preferred_element_type=jnp.float32)
    n1 = jnp.sum(mask1, axis=0, keepdims=True)
    loc2 = jnp.dot(tri, mask2, preferred_element_type=jnp.float32) + n1

    me = jnp.sum(gates, axis=0, keepdims=True) / S
    ce = jnp.sum(mask1, axis=0, keepdims=True) / S
    laux_ref[...] = jnp.sum(me * ce, axis=1, keepdims=True) * E

    keep1 = (loc1 < C).astype(jnp.float32) * mask1
    keep2 = (loc2 < C).astype(jnp.float32) * mask2

    c1 = jnp.sum(loc1 * keep1, axis=1, keepdims=True).astype(jnp.int32)
    c2 = jnp.sum(loc2 * keep2, axis=1, keepdims=True).astype(jnp.int32)
    valid1 = jnp.sum(keep1, axis=1, keepdims=True) > 0.0
    valid2 = jnp.sum(keep2, axis=1, keepdims=True) > 0.0

    g1s = jnp.sum(gates * keep1, axis=1, keepdims=True)
    g2s = jnp.sum(gates * keep2, axis=1, keepdims=True)
    denom = jnp.maximum(g1s + g2s, jnp.finfo(jnp.float32).eps)
    g1 = g1s / denom
    g2 = g2s / denom

    def grow(e_idx, c_idx, valid):
        g = ((c_idx >> 6) << 9) + (e_idx << 6) + (c_idx & 63)
        return jnp.where(valid, g, DUMP)

    gs1 = grow(idx1, c1, valid1)
    gs2 = grow(idx2, c2, valid2)
    r1 = jnp.minimum(gs1, NSLOT - 1)
    r2 = jnp.minimum(gs2, NSLOT - 1)

    zero = jnp.zeros((S, 1), jnp.float32)
    meta = jnp.concatenate(
        [gs1.astype(jnp.float32), gs2.astype(jnp.float32),
         r1.astype(jnp.float32), r2.astype(jnp.float32),
         g1, g2, zero, zero], axis=1)
    meta_ref[...] = meta


def _gating(x2d, wg):
    return pl.pallas_call(
        _gating_body,
        out_shape=(jax.ShapeDtypeStruct((S, E), jnp.float32),
                   jax.ShapeDtypeStruct((1, 1), jnp.float32)),
    )(x2d, wg)


def _dispatch_body(x_hbm, sidx_hbm, out_hbm, rows_v, i1_v, i2_v, sem):
    cid = lax.axis_index("c")
    sid = lax.axis_index("s")
    wid = sid * 2 + cid
    base = wid * (S // 32)
    pltpu.sync_copy(sidx_hbm.at[wid, 0], i1_v)
    pltpu.sync_copy(sidx_hbm.at[wid, 1], i2_v)
    pltpu.sync_copy(x_hbm.at[pl.ds(base, S // 32)], rows_v)
    c1 = pltpu.async_copy(rows_v, out_hbm.at[i1_v], sem)
    c2 = pltpu.async_copy(rows_v, out_hbm.at[i2_v], sem)
    c1.wait()
    c2.wait()


def _dispatch_scatter(x2d, sidx):
    mesh = plsc.VectorSubcoreMesh(core_axis_name="c", subcore_axis_name="s")
    return pl.kernel(
        _dispatch_body,
        mesh=mesh,
        out_type=jax.ShapeDtypeStruct((TOK_TAB // 1, M), jnp.float32),
        scratch_types=[
            pltpu.VMEM((S // 32, M), jnp.float32),
            pltpu.VMEM((S // 32,), jnp.int32),
            pltpu.VMEM((S // 32,), jnp.int32),
            pltpu.SemaphoreType.DMA,
        ],
    )(x2d, sidx)


def _combine_body(eo_hbm, ridx_hbm, g1r_hbm, g2r_hbm, out_hbm,
                  ii_v, g1_v, g2_v,
                  r0, r1_, o0, o1,
                  gsem0, gsem1, osem0, osem1):
    cid = lax.axis_index("c")
    sid = lax.axis_index("s")
    wid = sid * 2 + cid
    ntok = S // 32
    ck = 16
    nch = ntok // ck
    base = wid * ntok
    pltpu.sync_copy(ridx_hbm.at[wid], ii_v)
    pltpu.sync_copy(g1r_hbm.at[wid], g1_v)
    pltpu.sync_copy(g2r_hbm.at[wid], g2_v)
    rows = (r0, r1_)
    ov = (o0, o1)
    gsem = (gsem0, gsem1)
    osem = (osem0, osem1)

    def gather(c, b):
        return pltpu.async_copy(
            eo_hbm.at[ii_v.at[pl.ds(c * 2 * ck, 2 * ck)]], rows[b], gsem[b])

    pend = [gather(0, 0), gather(1, 1)]
    outc = [None, None]
    for c in range(nch):
        b = c % 2
        pend[b].wait()
        if outc[b] is not None:
            outc[b].wait()
        for t in range(ck):
            g1b = g1_v[c * ck + t]
            g2b = g2_v[c * ck + t]
            m1 = g1b > 0.0
            m2 = g2b > 0.0
            zero = jnp.zeros((16,), jnp.float32)

            def fma(k, _):
                for kk in range(8):
                    off = k * 128 + kk * 16
                    t1 = jnp.where(
                        m1, rows[b][2 * t, pl.ds(off, 16)] * g1b, zero)
                    t2 = jnp.where(
                        m2, rows[b][2 * t + 1, pl.ds(off, 16)] * g2b, zero)
                    ov[b][t, pl.ds(off, 16)] = t1 + t2
                return 0
            lax.fori_loop(0, M // 128, fma, 0)
        outc[b] = pltpu.async_copy(
            ov[b], out_hbm.at[pl.ds(base + c * ck, ck)], osem[b])
        if c + 2 < nch:
            pend[b] = gather(c + 2, b)
    for oc in outc:
        if oc is not None:
            oc.wait()


def _combine_sc(eo, ridx, g1r, g2r):
    mesh = plsc.VectorSubcoreMesh(core_axis_name="c", subcore_axis_name="s")
    ck = 16
    return pl.kernel(
        _combine_body,
        mesh=mesh,
        out_type=jax.ShapeDtypeStruct((S, M), jnp.float32),
        scratch_types=[
            pltpu.VMEM((2 * (S // 32),), jnp.int32),
            pltpu.VMEM((S // 32, 16), jnp.float32),
            pltpu.VMEM((S // 32, 16), jnp.float32),
            pltpu.VMEM((2 * ck, M), jnp.float32),
            pltpu.VMEM((2 * ck, M), jnp.float32),
            pltpu.VMEM((ck, M), jnp.float32),
            pltpu.VMEM((ck, M), jnp.float32),
            pltpu.SemaphoreType.DMA,
            pltpu.SemaphoreType.DMA,
            pltpu.SemaphoreType.DMA,
            pltpu.SemaphoreType.DMA,
        ],
    )(eo, ridx, g1r, g2r)


def _ffn_body(x_ref, w1_ref, b1_ref, w2_ref, b2_ref, o_ref):
    h = jnp.dot(x_ref[...], w1_ref[0], preferred_element_type=jnp.float32)
    h = jnp.maximum(h + b1_ref[0], 0.0)
    o = jnp.dot(h, w2_ref[0], preferred_element_type=jnp.float32)
    o_ref[...] = o + b2_ref[0]


def _ffn(xg, w1, b1, w2, b2):
    return pl.pallas_call(
        _ffn_body,
        grid=(NCHUNK,),
        in_specs=[
            pl.BlockSpec((C, M), lambda j: (j, 0)),
            pl.BlockSpec((1, M, F), lambda j: (j, 0, 0)),
            pl.BlockSpec((1, 1, F), lambda j: (j, 0, 0)),
            pl.BlockSpec((1, F, M), lambda j: (j, 0, 0)),
            pl.BlockSpec((1, 1, M), lambda j: (j, 0, 0)),
        ],
        out_specs=pl.BlockSpec((C, M), lambda j: (j, 0)),
        out_shape=jax.ShapeDtypeStruct((NSLOT, M), jnp.float32),
        compiler_params=pltpu.CompilerParams(
            dimension_semantics=("arbitrary",)),
    )(xg, w1, b1, w2, b2)


def kernel(x, wg, w1, b1, w2, b2):
    B = x.shape[0]
    x2d = x.reshape(S, M)

    meta, laux = _gating(x2d, wg)
    gs1 = meta[:, 0].astype(jnp.int32)
    gs2 = meta[:, 1].astype(jnp.int32)
    r1 = meta[:, 2].astype(jnp.int32)
    r2 = meta[:, 3].astype(jnp.int32)
    g1 = meta[:, 4:5]
    g2 = meta[:, 5:6]

    sidx = jnp.stack([gs1, gs2]).reshape(2, 32, S // 32).transpose(1, 0, 2)
    xg = _dispatch_scatter(x2d, sidx)

    eo = _ffn(xg, w1, b1.reshape(E, 1, F), w2, b2.reshape(E, 1, M))

    ridx = jnp.stack([r1, r2], axis=1).reshape(32, 2 * (S // 32))
    g1r = jnp.broadcast_to(g1, (S, 16)).reshape(32, S // 32, 16)
    g2r = jnp.broadcast_to(g2, (S, 16)).reshape(32, S // 32, 16)
    out = _combine_sc(eo, ridx, g1r, g2r)

    return out.reshape(B, S, M), laux.reshape(())
